# baseline (speedup 1.0000x reference)
from math import sqrt

import jax
import jax.numpy as jnp
import numpy as np
from jax import lax
from jax.experimental import pallas as pl
from jax.experimental.pallas import tpu as pltpu
from jax.experimental.pallas import tpu_sc as plsc

D_MODEL = 768
SEQ = 2048
BATCH = 4
ROWS = BATCH * SEQ

_INFO = plsc.get_sparse_core_info()
NC, NS, L = _INFO.num_cores, _INFO.num_subcores, _INFO.num_lanes
NW = NC * NS
ROWS_PER_W = ROWS // NW
CHUNK = 64
NCHUNK = ROWS_PER_W // CHUNK
VPR = D_MODEL // L
SCALE = np.float32(sqrt(D_MODEL))


def _positional_encoding(max_position, d_model):
    positions = np.arange(max_position)[:, None].astype(np.float64)
    dims = np.arange(d_model)[None, :].astype(np.float64)
    angle_rates = 1.0 / np.power(10000.0, (2.0 * (dims // 2)) / float(d_model))
    angles = positions * angle_rates
    pe = np.zeros((max_position, d_model), dtype=np.float64)
    pe[:, 0::2] = np.sin(angles[:, 0::2])
    pe[:, 1::2] = np.cos(angles[:, 1::2])
    return pe.astype(np.float32)


_PE = _positional_encoding(SEQ, D_MODEL)


def _sc_body(x_hbm, emb_hbm, pe_hbm, out_hbm, idx_v, rows_v, pe_v, sem):
    wid = lax.axis_index("s") * NC + lax.axis_index("c")
    base = wid * ROWS_PER_W
    s0 = base % SEQ
    for ch in range(NCHUNK):
        r0 = base + ch * CHUNK
        pltpu.sync_copy(x_hbm.at[pl.ds(r0, CHUNK)], idx_v)
        gather = pltpu.async_copy(emb_hbm.at[idx_v], rows_v, sem)
        pltpu.sync_copy(pe_hbm.at[pl.ds(s0 + ch * CHUNK, CHUNK)], pe_v)
        gather.wait()

        def row_body(r, carry):
            for c in range(VPR):
                sl = pl.ds(c * L, L)
                rows_v[r, sl] = rows_v[r, sl] * SCALE + pe_v[r, sl]
            return carry

        lax.fori_loop(0, CHUNK, row_body, 0)
        pltpu.sync_copy(rows_v, out_hbm.at[pl.ds(r0, CHUNK)])


@jax.jit
def _encoder(x_flat, embedding, pe):
    mesh = plsc.VectorSubcoreMesh(core_axis_name="c", subcore_axis_name="s")
    f = pl.kernel(
        _sc_body,
        out_type=jax.ShapeDtypeStruct((ROWS, D_MODEL), jnp.float32),
        mesh=mesh,
        scratch_types=[
            pltpu.VMEM((CHUNK,), jnp.int32),
            pltpu.VMEM((CHUNK, D_MODEL), jnp.float32),
            pltpu.VMEM((CHUNK, D_MODEL), jnp.float32),
            pltpu.SemaphoreType.DMA,
        ],
    )
    return f(x_flat, embedding, pe)


def kernel(x, embedding):
    x_flat = x.reshape(ROWS).astype(jnp.int32)
    out = _encoder(x_flat, embedding, _PE)
    return out.reshape(BATCH, SEQ, D_MODEL)
